# baseline (speedup 1.0000x reference)
import functools

import jax
import jax.numpy as jnp
from jax import lax
from jax.experimental import pallas as pl
from jax.experimental.pallas import tpu as pltpu

N_NODES = 10000
N_EDGES = 320000
D = 128
EPS = 1e-5

E_BLK = 2000


def _dot(a, b):
    return jax.lax.dot_general(a, b, (((1,), (0,)), ((), ())),
                               preferred_element_type=jnp.float32)


def _prep_body(h_in, embW, embb, V, U, B, C, h0, Vh, Uh, Bh, Ch):
    x = _dot(h_in[...], embW[...]) + embb[...][None, :]
    h0[...] = x
    Vh[...] = _dot(x, V[...])
    Uh[...] = _dot(x, U[...])
    Bh[...] = _dot(x, B[...])
    Ch[...] = _dot(x, C[...])


def _prep(h_in, embW, embb, V, U, B, C):
    n = h_in.shape[0]
    out = [jax.ShapeDtypeStruct((n, D), jnp.float32)] * 5
    return pl.pallas_call(_prep_body, out_shape=out)(h_in, embW, embb, V, U, B, C)


def _edge_embed_body(e_in, W, b, e0):
    e0[...] = _dot(e_in[...], W[...]) + b[...][None, :]


def _edge_embed(e_in, W, b):
    ne, fe = e_in.shape
    grid = ne // E_BLK
    return pl.pallas_call(
        _edge_embed_body,
        grid=(grid,),
        in_specs=[
            pl.BlockSpec((E_BLK, fe), lambda i: (i, 0)),
            pl.BlockSpec((fe, D), lambda i: (0, 0)),
            pl.BlockSpec((D,), lambda i: (0,)),
        ],
        out_specs=pl.BlockSpec((E_BLK, D), lambda i: (i, 0)),
        out_shape=jax.ShapeDtypeStruct((ne, D), jnp.float32),
    )(e_in, W, b)


def _node_update_body(h, Uh, agg, mats, outs):
    x = Uh[...] + agg[...]
    mu = jnp.mean(x, axis=0)
    var = jnp.mean(x * x, axis=0) - mu * mu
    y = h[...] + jnp.maximum((x - mu[None, :]) * jax.lax.rsqrt(var + EPS)[None, :], 0.0)
    outs[0][...] = y
    for m_ref, o_ref in zip(mats, outs[1:]):
        o_ref[...] = _dot(y, m_ref[...])


def _node_update(h, Uh, agg, mats):
    n = h.shape[0]
    nmat = len(mats)

    def body(*refs):
        _node_update_body(refs[0], refs[1], refs[2], refs[3:3 + nmat],
                          refs[3 + nmat:])

    out = [jax.ShapeDtypeStruct((n, D), jnp.float32)] * (1 + nmat)
    return pl.pallas_call(body, out_shape=out)(h, Uh, agg, *mats)


def _edge_stats_body(e0, g, A, out, acc):
    i = pl.program_id(0)
    t = _dot(e0[...], A[...]) + g[...]
    s = jnp.sum(t, axis=0)
    s2 = jnp.sum(t * t, axis=0)

    @pl.when(i == 0)
    def _():
        acc[...] = jnp.zeros_like(acc)

    acc[0, :] += s
    acc[1, :] += s2

    @pl.when(i == pl.num_programs(0) - 1)
    def _():
        out[...] = acc[...]


def _edge_stats(e0, g, A):
    ne = e0.shape[0]
    grid = ne // E_BLK
    return pl.pallas_call(
        _edge_stats_body,
        grid=(grid,),
        in_specs=[
            pl.BlockSpec((E_BLK, D), lambda i: (i, 0)),
            pl.BlockSpec((E_BLK, D), lambda i: (i, 0)),
            pl.BlockSpec((D, D), lambda i: (0, 0)),
        ],
        out_specs=pl.BlockSpec((8, D), lambda i: (0, 0)),
        out_shape=jax.ShapeDtypeStruct((8, D), jnp.float32),
        scratch_shapes=[pltpu.VMEM((8, D), jnp.float32)],
    )(e0, g, A)


def _edge_update_body(e0, g, A, stats, e1):
    inv_n = 1.0 / N_EDGES
    mu = stats[0, :] * inv_n
    var = stats[1, :] * inv_n - mu * mu
    t = _dot(e0[...], A[...]) + g[...]
    y = (t - mu[None, :]) * jax.lax.rsqrt(var + EPS)[None, :]
    e1[...] = e0[...] + jnp.maximum(y, 0.0)


def _edge_update(e0, g, A, stats):
    ne = e0.shape[0]
    grid = ne // E_BLK
    return pl.pallas_call(
        _edge_update_body,
        grid=(grid,),
        in_specs=[
            pl.BlockSpec((E_BLK, D), lambda i: (i, 0)),
            pl.BlockSpec((E_BLK, D), lambda i: (i, 0)),
            pl.BlockSpec((D, D), lambda i: (0, 0)),
            pl.BlockSpec((8, D), lambda i: (0, 0)),
        ],
        out_specs=pl.BlockSpec((E_BLK, D), lambda i: (i, 0)),
        out_shape=jax.ShapeDtypeStruct((ne, D), jnp.float32),
    )(e0, g, A, stats)


def _final_head_body(h, W1a, W1b, W1c, b1, r_out, P, Q):
    y = h[...]
    moy = jnp.mean(y, axis=0)
    r = _dot(moy[None, :], W1a[...]) + b1[...][None, :]
    r_out[...] = jnp.broadcast_to(r, r_out.shape)
    P[...] = _dot(y, W1b[...])
    Q[...] = _dot(y, W1c[...])


def _final_head(h, W1a, W1b, W1c, b1):
    n = h.shape[0]
    out = [
        jax.ShapeDtypeStruct((8, D), jnp.float32),
        jax.ShapeDtypeStruct((n, D), jnp.float32),
        jax.ShapeDtypeStruct((n, D), jnp.float32),
    ]
    return pl.pallas_call(_final_head_body, out_shape=out)(h, W1a, W1b, W1c, b1)


def _gather_pair_add(Bh, Ch, src, dst):
    return Bh[dst] + Ch[src]


def _message_segmax(Vh, e, src, dst):
    m = Vh[src] * jax.nn.sigmoid(e)
    agg = jax.ops.segment_max(m, dst, num_segments=N_NODES)
    deg = jax.ops.segment_sum(jnp.ones((src.shape[0],), jnp.float32), dst,
                              num_segments=N_NODES)
    return jnp.where((deg > 0)[:, None], agg, 0.0)


def _final_edge(r, P, Q, W2, b2, src, dst):
    v = r[0, :][None, :] + P[src] + Q[dst]
    z = jnp.maximum(v, 0.0) @ W2 + b2
    return jax.nn.sigmoid(z[:, 0])


def kernel(h, e, edge_index, params):
    src = edge_index[0]
    dst = edge_index[1]

    h0, Vh0, Uh0, Bh0, Ch0 = _prep(
        h, params['emb_n_W'], params['emb_n_b'],
        params['V0'], params['U0'], params['B0'], params['C0'])
    e0 = _edge_embed(e, params['emb_e_W'], params['emb_e_b'])

    agg0 = _message_segmax(Vh0, e0, src, dst)
    g0 = _gather_pair_add(Bh0, Ch0, src, dst)
    h1, Vh1, Uh1 = _node_update(h0, Uh0, agg0, [params['V1'], params['U1']])
    stats0 = _edge_stats(e0, g0, params['A0'])
    e1 = _edge_update(e0, g0, params['A0'], stats0)

    agg1 = _message_segmax(Vh1, e1, src, dst)
    W1 = params['W1_W']
    (h2,) = _node_update(h1, Uh1, agg1, [])

    r, P, Q = _final_head(h2, W1[:D], W1[D:2 * D], W1[2 * D:], params['W1_b'])
    return _final_edge(r, P, Q, params['W2_W'], params['W2_b'], src, dst)


# speedup vs baseline: 1.7777x; 1.7777x over previous
import functools

import jax
import jax.numpy as jnp
from jax import lax
from jax.experimental import pallas as pl
from jax.experimental.pallas import tpu as pltpu
from jax.experimental.pallas import tpu_sc as plsc

N_NODES = 10000
N_EDGES = 320000
D = 128
EPS = 1e-5

E_BLK = 2000


def _dot(a, b):
    return jax.lax.dot_general(a, b, (((1,), (0,)), ((), ())),
                               preferred_element_type=jnp.float32)


def _prep_body(h_in, embW, embb, V, U, B, C, h0, Vh, Uh, Bh, Ch):
    x = _dot(h_in[...], embW[...]) + embb[...][None, :]
    h0[...] = x
    Vh[...] = _dot(x, V[...])
    Uh[...] = _dot(x, U[...])
    Bh[...] = _dot(x, B[...])
    Ch[...] = _dot(x, C[...])


def _prep(h_in, embW, embb, V, U, B, C):
    n = h_in.shape[0]
    out = [jax.ShapeDtypeStruct((n, D), jnp.float32)] * 5
    return pl.pallas_call(_prep_body, out_shape=out)(h_in, embW, embb, V, U, B, C)


def _edge_embed_body(e_in, W, b, e0):
    e0[...] = _dot(e_in[...], W[...]) + b[...][None, :]


def _edge_embed(e_in, W, b):
    ne, fe = e_in.shape
    grid = ne // E_BLK
    return pl.pallas_call(
        _edge_embed_body,
        grid=(grid,),
        in_specs=[
            pl.BlockSpec((E_BLK, fe), lambda i: (i, 0)),
            pl.BlockSpec((fe, D), lambda i: (0, 0)),
            pl.BlockSpec((D,), lambda i: (0,)),
        ],
        out_specs=pl.BlockSpec((E_BLK, D), lambda i: (i, 0)),
        out_shape=jax.ShapeDtypeStruct((ne, D), jnp.float32),
    )(e_in, W, b)


def _node_update_body(h, Uh, agg, mats, outs):
    x = Uh[...] + agg[...]
    mu = jnp.mean(x, axis=0)
    var = jnp.mean(x * x, axis=0) - mu * mu
    y = h[...] + jnp.maximum((x - mu[None, :]) * jax.lax.rsqrt(var + EPS)[None, :], 0.0)
    outs[0][...] = y
    for m_ref, o_ref in zip(mats, outs[1:]):
        o_ref[...] = _dot(y, m_ref[...])


def _node_update(h, Uh, agg, mats):
    n = h.shape[0]
    nmat = len(mats)

    def body(*refs):
        _node_update_body(refs[0], refs[1], refs[2], refs[3:3 + nmat],
                          refs[3 + nmat:])

    out = [jax.ShapeDtypeStruct((n, D), jnp.float32)] * (1 + nmat)
    return pl.pallas_call(body, out_shape=out)(h, Uh, agg, *mats)


def _edge_stats_body(e0, g, A, out, acc):
    i = pl.program_id(0)
    t = _dot(e0[...], A[...]) + g[...]
    s = jnp.sum(t, axis=0)
    s2 = jnp.sum(t * t, axis=0)

    @pl.when(i == 0)
    def _():
        acc[...] = jnp.zeros_like(acc)

    acc[0, :] += s
    acc[1, :] += s2

    @pl.when(i == pl.num_programs(0) - 1)
    def _():
        out[...] = acc[...]


def _edge_stats(e0, g, A):
    ne = e0.shape[0]
    grid = ne // E_BLK
    return pl.pallas_call(
        _edge_stats_body,
        grid=(grid,),
        in_specs=[
            pl.BlockSpec((E_BLK, D), lambda i: (i, 0)),
            pl.BlockSpec((E_BLK, D), lambda i: (i, 0)),
            pl.BlockSpec((D, D), lambda i: (0, 0)),
        ],
        out_specs=pl.BlockSpec((8, D), lambda i: (0, 0)),
        out_shape=jax.ShapeDtypeStruct((8, D), jnp.float32),
        scratch_shapes=[pltpu.VMEM((8, D), jnp.float32)],
    )(e0, g, A)


def _edge_update_body(e0, g, A, stats, e1):
    inv_n = 1.0 / N_EDGES
    mu = stats[0, :] * inv_n
    var = stats[1, :] * inv_n - mu * mu
    t = _dot(e0[...], A[...]) + g[...]
    y = (t - mu[None, :]) * jax.lax.rsqrt(var + EPS)[None, :]
    e1[...] = e0[...] + jnp.maximum(y, 0.0)


def _edge_update(e0, g, A, stats):
    ne = e0.shape[0]
    grid = ne // E_BLK
    return pl.pallas_call(
        _edge_update_body,
        grid=(grid,),
        in_specs=[
            pl.BlockSpec((E_BLK, D), lambda i: (i, 0)),
            pl.BlockSpec((E_BLK, D), lambda i: (i, 0)),
            pl.BlockSpec((D, D), lambda i: (0, 0)),
            pl.BlockSpec((8, D), lambda i: (0, 0)),
        ],
        out_specs=pl.BlockSpec((E_BLK, D), lambda i: (i, 0)),
        out_shape=jax.ShapeDtypeStruct((ne, D), jnp.float32),
    )(e0, g, A, stats)


def _final_head_body(h, W1a, W1b, W1c, b1, r_out, P, Q):
    y = h[...]
    moy = jnp.mean(y, axis=0)
    r = _dot(moy[None, :], W1a[...]) + b1[...][None, :]
    r_out[...] = jnp.broadcast_to(r, r_out.shape)
    P[...] = _dot(y, W1b[...])
    Q[...] = _dot(y, W1c[...])


def _final_head(h, W1a, W1b, W1c, b1):
    n = h.shape[0]
    out = [
        jax.ShapeDtypeStruct((8, D), jnp.float32),
        jax.ShapeDtypeStruct((n, D), jnp.float32),
        jax.ShapeDtypeStruct((n, D), jnp.float32),
    ]
    return pl.pallas_call(_final_head_body, out_shape=out)(h, W1a, W1b, W1c, b1)


_NW = 32
_PER_W = N_EDGES // _NW
_KG = 128
_NFULL = _PER_W // _KG
_TAIL = _PER_W - _NFULL * _KG


def _message_rows_gather(Vh, e, src, dst, Bh, Ch):
    mesh = plsc.VectorSubcoreMesh(core_axis_name="c", subcore_axis_name="s")

    @functools.partial(
        pl.kernel,
        out_type=(jax.ShapeDtypeStruct((N_EDGES, D), jnp.float32),
                  jax.ShapeDtypeStruct((N_EDGES, D), jnp.float32)),
        mesh=mesh,
        scratch_types=[
            pltpu.VMEM((_KG,), jnp.int32),
            pltpu.VMEM((_KG,), jnp.int32),
            pltpu.VMEM((_KG, D), jnp.float32),
            pltpu.VMEM((_KG, D), jnp.float32),
            pltpu.VMEM((_KG, D), jnp.float32),
            pltpu.SemaphoreType.DMA,
            pltpu.SemaphoreType.DMA,
            pltpu.SemaphoreType.DMA,
        ],
    )
    def sc_message_gather(vh_h, e_h, src_h, dst_h, bh_h, ch_h, m_h, g_h,
                          idx_s, idx_d, erows, vrows, grows,
                          sem_v, sem_e, sem_b):
        wid = lax.axis_index("s") * 2 + lax.axis_index("c")
        base_w = wid * _PER_W

        def chunk(base, n):
            base = pl.multiple_of(base, 8)
            pltpu.sync_copy(src_h.at[pl.ds(base, n)], idx_s.at[pl.ds(0, n)])
            pltpu.sync_copy(dst_h.at[pl.ds(base, n)], idx_d.at[pl.ds(0, n)])
            cp_e = pltpu.async_copy(e_h.at[pl.ds(base, n)],
                                    erows.at[pl.ds(0, n)], sem_e)
            cp_v = pltpu.async_copy(vh_h.at[idx_s.at[pl.ds(0, n)]],
                                    vrows.at[pl.ds(0, n)], sem_v)
            cp_b = pltpu.async_copy(bh_h.at[idx_d.at[pl.ds(0, n)]],
                                    grows.at[pl.ds(0, n)], sem_b)
            cp_b.wait()
            cp_c = pltpu.async_copy(ch_h.at[idx_s.at[pl.ds(0, n)]],
                                    grows.at[pl.ds(0, n)], sem_b, add=True)
            cp_e.wait()
            cp_v.wait()

            def edge_body(i, c):
                for j in range(D // 16):
                    ev = erows[i, pl.ds(j * 16, 16)]
                    sig = 1.0 / (1.0 + jnp.exp(-ev))
                    erows[i, pl.ds(j * 16, 16)] = (
                        vrows[i, pl.ds(j * 16, 16)] * sig)
                return c

            lax.fori_loop(0, n, edge_body, 0)
            pltpu.sync_copy(erows.at[pl.ds(0, n)], m_h.at[pl.ds(base, n)])
            cp_c.wait()
            pltpu.sync_copy(grows.at[pl.ds(0, n)], g_h.at[pl.ds(base, n)])

        def body(i, carry):
            chunk(base_w + i * _KG, _KG)
            return carry

        lax.fori_loop(0, _NFULL, body, 0)
        chunk(base_w + _NFULL * _KG, _TAIL)

    return sc_message_gather(Vh, e, src, dst, Bh, Ch)


def _message_rows(Vh, e, src):
    mesh = plsc.VectorSubcoreMesh(core_axis_name="c", subcore_axis_name="s")

    @functools.partial(
        pl.kernel,
        out_type=jax.ShapeDtypeStruct((N_EDGES, D), jnp.float32),
        mesh=mesh,
        scratch_types=[
            pltpu.VMEM((_KG,), jnp.int32),
            pltpu.VMEM((_KG, D), jnp.float32),
            pltpu.VMEM((_KG, D), jnp.float32),
            pltpu.SemaphoreType.DMA,
        ],
    )
    def sc_message_rows(vh_h, e_h, src_h, out_h, idx_s, vrows, erows, sem):
        wid = lax.axis_index("s") * 2 + lax.axis_index("c")
        base_w = wid * _PER_W

        def chunk(base, n):
            base = pl.multiple_of(base, 8)
            pltpu.sync_copy(src_h.at[pl.ds(base, n)], idx_s.at[pl.ds(0, n)])
            cp_v = pltpu.async_copy(vh_h.at[idx_s.at[pl.ds(0, n)]],
                                    vrows.at[pl.ds(0, n)], sem)
            pltpu.sync_copy(e_h.at[pl.ds(base, n)], erows.at[pl.ds(0, n)])
            cp_v.wait()

            def edge_body(i, c):
                for j in range(D // 16):
                    ev = erows[i, pl.ds(j * 16, 16)]
                    sig = 1.0 / (1.0 + jnp.exp(-ev))
                    erows[i, pl.ds(j * 16, 16)] = (
                        vrows[i, pl.ds(j * 16, 16)] * sig)
                return c

            lax.fori_loop(0, n, edge_body, 0)
            pltpu.sync_copy(erows.at[pl.ds(0, n)], out_h.at[pl.ds(base, n)])

        def body(i, carry):
            chunk(base_w + i * _KG, _KG)
            return carry

        lax.fori_loop(0, _NFULL, body, 0)
        chunk(base_w + _NFULL * _KG, _TAIL)

    return sc_message_rows(Vh, e, src)


def _final_gather_relu(r, P, Q, src, dst):
    mesh = plsc.VectorSubcoreMesh(core_axis_name="c", subcore_axis_name="s")

    @functools.partial(
        pl.kernel,
        out_type=jax.ShapeDtypeStruct((N_EDGES, D), jnp.float32),
        mesh=mesh,
        scratch_types=[
            pltpu.VMEM((_KG,), jnp.int32),
            pltpu.VMEM((_KG,), jnp.int32),
            pltpu.VMEM((_KG, D), jnp.float32),
            pltpu.VMEM((_KG, D), jnp.float32),
            pltpu.VMEM((8, D), jnp.float32),
            pltpu.SemaphoreType.DMA,
            pltpu.SemaphoreType.DMA,
        ],
    )
    def sc_final_rows(p_h, q_h, r_h, src_h, dst_h, out_h,
                      idx_s, idx_d, rows, qrows, r_v, sem, sem_q):
        wid = lax.axis_index("s") * 2 + lax.axis_index("c")
        base_w = wid * _PER_W
        pltpu.sync_copy(r_h, r_v)
        rj = [r_v[0, pl.ds(j * 16, 16)] for j in range(D // 16)]

        def chunk(base, n):
            base = pl.multiple_of(base, 8)
            pltpu.sync_copy(src_h.at[pl.ds(base, n)], idx_s.at[pl.ds(0, n)])
            pltpu.sync_copy(dst_h.at[pl.ds(base, n)], idx_d.at[pl.ds(0, n)])
            cp_p = pltpu.async_copy(p_h.at[idx_s.at[pl.ds(0, n)]],
                                    rows.at[pl.ds(0, n)], sem)
            cp_q = pltpu.async_copy(q_h.at[idx_d.at[pl.ds(0, n)]],
                                    qrows.at[pl.ds(0, n)], sem_q)
            cp_p.wait()
            cp_q.wait()

            def edge_body(i, c):
                for j in range(D // 16):
                    rows[i, pl.ds(j * 16, 16)] = jnp.maximum(
                        rows[i, pl.ds(j * 16, 16)]
                        + qrows[i, pl.ds(j * 16, 16)] + rj[j], 0.0)
                return c

            lax.fori_loop(0, n, edge_body, 0)
            pltpu.sync_copy(rows.at[pl.ds(0, n)], out_h.at[pl.ds(base, n)])

        def body(i, carry):
            chunk(base_w + i * _KG, _KG)
            return carry

        lax.fori_loop(0, _NFULL, body, 0)
        chunk(base_w + _NFULL * _KG, _TAIL)

    return sc_final_rows(P, Q, r, src, dst)


_NPW = 320
_CSC = 2000
_NB16 = _CSC // 16
_PCAP = _CSC + 256
_NEG = -3.0e38


def _message_segmax(Vh, e, src, dst):
    mesh = plsc.VectorSubcoreMesh(core_axis_name="c", subcore_axis_name="s")

    @functools.partial(
        pl.kernel,
        out_type=jax.ShapeDtypeStruct((_NW * _NPW, D), jnp.float32),
        mesh=mesh,
        scratch_types=[
            pltpu.VMEM((_CSC,), jnp.int32),
            pltpu.VMEM((_PCAP,), jnp.int32),
            pltpu.VMEM((_PCAP,), jnp.int32),
            pltpu.VMEM((_KG, D), jnp.float32),
            pltpu.VMEM((_KG, D), jnp.float32),
            pltpu.VMEM((_KG,), jnp.int32),
            pltpu.VMEM((_NPW + 1, D), jnp.float32),
            pltpu.SemaphoreType.DMA,
        ],
    )
    def sc_segmax(vh_h, e_h, src_h, dst_h, out_h,
                  dchunk, p_eid, p_off, erows, vrows, sbuf, tab, sem):
        wid = lax.axis_index("s") * 2 + lax.axis_index("c")
        lo = pl.multiple_of(wid * _NPW, 8)

        neg = jnp.full((16,), _NEG, jnp.float32)

        def init_row(i, c):
            for j in range(D // 16):
                tab[i, pl.ds(j * 16, 16)] = neg
            return c

        lax.fori_loop(0, _NPW + 1, init_row, 0)

        iota16 = lax.iota(jnp.int32, 16)

        def drain(q):
            q = pl.multiple_of(q, _KG)
            idx = p_eid.at[pl.ds(q, _KG)]
            pltpu.async_copy(e_h.at[idx], erows, sem).wait()
            pltpu.async_copy(src_h.at[idx], sbuf, sem).wait()
            pltpu.async_copy(vh_h.at[sbuf], vrows, sem).wait()

            def group_body(g, c):
                offs = p_off[pl.ds(pl.multiple_of(q + g * 16, 16), 16)]
                for t in range(16):
                    off = offs[t]
                    i = g * 16 + t
                    for j in range(D // 16):
                        ev = erows[i, pl.ds(j * 16, 16)]
                        sig = 1.0 / (1.0 + jnp.exp(-ev))
                        m = vrows[i, pl.ds(j * 16, 16)] * sig
                        tt = tab[off, pl.ds(j * 16, 16)]
                        tab[off, pl.ds(j * 16, 16)] = jnp.maximum(tt, m)
                return c

            lax.fori_loop(0, _KG // 16, group_body, 0)

        def chunk_body(ci, cnt):
            pltpu.sync_copy(dst_h.at[pl.ds(pl.multiple_of(ci * _CSC, 8), _CSC)],
                            dchunk)

            def blk_body(b, cnt):
                d = dchunk[pl.ds(pl.multiple_of(b * 16, 16), 16)]
                mask = (d >= lo) & (d < lo + _NPW)
                eid = iota16 + (ci * _CSC + b * 16)
                msel = mask.astype(jnp.int32)
                pref = plsc.cumsum(msel)
                pos = cnt + pref - msel
                plsc.store_scatter(p_eid, [pos], eid, mask=mask)
                plsc.store_scatter(p_off, [pos], d - lo, mask=mask)
                return cnt + pref[15]

            cnt = lax.fori_loop(0, _NB16, blk_body, cnt)

            nb = cnt // _KG

            def bstep(b, c):
                @pl.when(b < nb)
                def _():
                    drain(b * _KG)
                return c

            lax.fori_loop(0, _PCAP // _KG, bstep, 0)

            @pl.when(nb > 0)
            def _():
                dd = pl.multiple_of(nb * _KG, _KG)
                for t in range(_KG // 16):
                    p_eid[pl.ds(t * 16, 16)] = p_eid[pl.ds(dd + t * 16, 16)]
                    p_off[pl.ds(t * 16, 16)] = p_off[pl.ds(dd + t * 16, 16)]

            return cnt - nb * _KG

        cnt = lax.fori_loop(0, N_EDGES // _CSC, chunk_body, 0)

        zero16 = jnp.zeros((16,), jnp.int32)
        trash16 = jnp.full((16,), _NPW, jnp.int32)
        for t in range(_KG // 16):
            pos = cnt + iota16 + (t * 16)
            plsc.store_scatter(p_eid, [pos], zero16,
                               mask=jnp.full((16,), True))
            plsc.store_scatter(p_off, [pos], trash16,
                               mask=jnp.full((16,), True))

        @pl.when(cnt > 0)
        def _():
            drain(0)

        def flush_row(i, c):
            for j in range(D // 16):
                v = tab[i, pl.ds(j * 16, 16)]
                tab[i, pl.ds(j * 16, 16)] = jnp.where(v > -1.0e37, v, 0.0)
            return c

        lax.fori_loop(0, _NPW, flush_row, 0)
        pltpu.sync_copy(tab.at[pl.ds(0, _NPW)], out_h.at[pl.ds(lo, _NPW)])

    return sc_segmax(Vh, e, src, dst)[:N_NODES]


def _segmax_from_rows(m, src, dst):
    agg = jax.ops.segment_max(m, dst, num_segments=N_NODES)
    deg = jax.ops.segment_sum(jnp.ones_like(src, jnp.float32), dst,
                              num_segments=N_NODES)
    return jnp.where((deg > 0)[:, None], agg, 0.0)


def _segmax_xla(Vh, e, src, dst):
    return _segmax_from_rows(_message_rows(Vh, e, src), src, dst)


def kernel(h, e, edge_index, params):
    src = edge_index[0]
    dst = edge_index[1]

    h0, Vh0, Uh0, Bh0, Ch0 = _prep(
        h, params['emb_n_W'], params['emb_n_b'],
        params['V0'], params['U0'], params['B0'], params['C0'])
    e0 = _edge_embed(e, params['emb_e_W'], params['emb_e_b'])

    m0, g0 = _message_rows_gather(Vh0, e0, src, dst, Bh0, Ch0)
    agg0 = _segmax_from_rows(m0, src, dst)
    h1, Vh1, Uh1 = _node_update(h0, Uh0, agg0, [params['V1'], params['U1']])
    stats0 = _edge_stats(e0, g0, params['A0'])
    e1 = _edge_update(e0, g0, params['A0'], stats0)

    agg1 = _segmax_xla(Vh1, e1, src, dst)
    W1 = params['W1_W']
    (h2,) = _node_update(h1, Uh1, agg1, [])

    r, P, Q = _final_head(h2, W1[:D], W1[D:2 * D], W1[2 * D:], params['W1_b'])
    rows = _final_gather_relu(r, P, Q, src, dst)
    z = rows @ params['W2_W'] + params['W2_b']
    return jax.nn.sigmoid(z[:, 0])
